# baseline (speedup 1.0000x reference)
import functools

import jax
import jax.numpy as jnp
from jax.experimental import pallas as pl
from jax.experimental.pallas import tpu as pltpu

_LOG2E = 1.4426950408889634


def _prepass_body(x_ref, w_ref, asrc_ref, adst_ref, wh_ref, s_ref,
                  dexp_ref, dexp2_ref, dmax_ref):
    i = pl.program_id(0)
    wh = jnp.dot(x_ref[...], w_ref[...], preferred_element_type=jnp.float32)
    wh_ref[...] = wh
    s_ref[...] = jnp.dot(wh, asrc_ref[...], preferred_element_type=jnp.float32)
    d = jnp.dot(wh, adst_ref[...], preferred_element_type=jnp.float32)
    dexp_ref[...] = jnp.exp(d).astype(jnp.bfloat16)
    dexp2_ref[...] = jnp.exp(0.2 * d).astype(jnp.bfloat16)
    bmax = jnp.max(d, axis=0, keepdims=True)

    @pl.when(i == 0)
    def _():
        dmax_ref[...] = bmax

    @pl.when(i > 0)
    def _():
        dmax_ref[...] = jnp.maximum(dmax_ref[...], bmax)


def _main_body(*refs, n, h_heads, d_dim, bm, bn, k_chunks):
    adj_refs = refs[:k_chunks]
    dt_ref, wh2_ref, s_ref, dmax_ref, out_ref, srow_ref = refs[k_chunks:]

    s = s_ref[...]
    x = s + dmax_ref[...]
    m = jnp.maximum(x, 0.2 * x)
    srow_ref[:, :h_heads] = jnp.exp(s - m).astype(jnp.bfloat16)
    srow_ref[:, h_heads:2 * h_heads] = jnp.exp(0.2 * s - m).astype(jnp.bfloat16)

    col_ids = jax.lax.broadcasted_iota(jnp.int32, (1, bn), 1)
    for k in range(k_chunks):
        rows = slice(k * bm, (k + 1) * bm)
        adjm = jnp.where(col_ids < n, adj_refs[k][...], 0.0).astype(jnp.bfloat16)

        for h in range(h_heads):
            u1 = srow_ref[rows, h:h + 1]
            u2 = srow_ref[rows, h_heads + h:h_heads + h + 1]
            v1 = dt_ref[h:h + 1, :]
            v2 = dt_ref[h_heads + h:h_heads + h + 1, :]
            p = jnp.maximum(u1 * v1, u2 * v2) * adjm
            part = jnp.dot(p, wh2_ref[:, 32 * h:32 * h + 32],
                           preferred_element_type=jnp.float32)
            y = part[:, :d_dim] / part[:, d_dim:d_dim + 1]
            out_ref[rows, h * d_dim:(h + 1) * d_dim] = \
                jnp.where(y > 0, y, jnp.exp(y) - 1.0)


def kernel(features, adj, W, a_src, a_dst):
    n, f = features.shape
    h_heads, _, d_dim = W.shape
    hd = h_heads * d_dim

    bm1 = 1000
    r1 = n // bm1
    w_cat = jnp.transpose(W, (1, 0, 2)).reshape(f, hd)
    asrc = jnp.zeros((hd, h_heads), jnp.float32)
    adst = jnp.zeros((hd, h_heads), jnp.float32)
    for h in range(h_heads):
        asrc = asrc.at[h * d_dim:(h + 1) * d_dim, h].set(a_src[h])
        adst = adst.at[h * d_dim:(h + 1) * d_dim, h].set(a_dst[h])

    wh, s, dexp, dexp2, dmax = pl.pallas_call(
        _prepass_body,
        grid=(r1,),
        in_specs=[
            pl.BlockSpec((bm1, f), lambda i: (i, 0)),
            pl.BlockSpec((f, hd), lambda i: (0, 0)),
            pl.BlockSpec((hd, h_heads), lambda i: (0, 0)),
            pl.BlockSpec((hd, h_heads), lambda i: (0, 0)),
        ],
        out_specs=[
            pl.BlockSpec((bm1, hd), lambda i: (i, 0)),
            pl.BlockSpec((bm1, h_heads), lambda i: (i, 0)),
            pl.BlockSpec((bm1, h_heads), lambda i: (i, 0)),
            pl.BlockSpec((bm1, h_heads), lambda i: (i, 0)),
            pl.BlockSpec((1, h_heads), lambda i: (0, 0)),
        ],
        out_shape=[
            jax.ShapeDtypeStruct((n, hd), jnp.float32),
            jax.ShapeDtypeStruct((n, h_heads), jnp.float32),
            jax.ShapeDtypeStruct((n, h_heads), jnp.bfloat16),
            jax.ShapeDtypeStruct((n, h_heads), jnp.bfloat16),
            jax.ShapeDtypeStruct((1, h_heads), jnp.float32),
        ],
    )(features, w_cat, asrc, adst)

    bm, kc = 80, 5
    bn = 128 * pl.cdiv(n, 128)
    npad = bn
    rr = pl.cdiv(n, bm * kc)

    dt_pad = jnp.zeros((8, npad), jnp.bfloat16)
    dt_pad = dt_pad.at[:h_heads, :n].set(dexp.T)
    dt_pad = dt_pad.at[h_heads:2 * h_heads, :n].set(dexp2.T)
    wh2_pad = jnp.zeros((npad, 128), jnp.bfloat16)
    for h in range(h_heads):
        wh2_pad = wh2_pad.at[:n, 32 * h:32 * h + d_dim].set(
            wh[:, h * d_dim:(h + 1) * d_dim])
        wh2_pad = wh2_pad.at[:n, 32 * h + d_dim].set(1.0)

    body = functools.partial(_main_body, n=n, h_heads=h_heads, d_dim=d_dim,
                             bm=bm, bn=bn, k_chunks=kc)
    adj_specs = [pl.BlockSpec((bm, bn), lambda r, k=k: (r * kc + k, 0))
                 for k in range(kc)]
    out = pl.pallas_call(
        body,
        grid=(rr,),
        in_specs=adj_specs + [
            pl.BlockSpec((8, bn), lambda r: (0, 0)),
            pl.BlockSpec((bn, 128), lambda r: (0, 0)),
            pl.BlockSpec((bm * kc, h_heads), lambda r: (r, 0)),
            pl.BlockSpec((1, h_heads), lambda r: (0, 0)),
        ],
        out_specs=pl.BlockSpec((bm * kc, hd), lambda r: (r, 0)),
        out_shape=jax.ShapeDtypeStruct((n, hd), jnp.float32),
        scratch_shapes=[
            pltpu.VMEM((bm * kc, 2 * h_heads), jnp.bfloat16),
        ],
        compiler_params=pltpu.CompilerParams(
            dimension_semantics=("parallel",),
        ),
    )(*([adj] * kc), dt_pad, wh2_pad, s, dmax)
    return out


# speedup vs baseline: 1.3461x; 1.3461x over previous
import functools

import jax
import jax.numpy as jnp
from jax.experimental import pallas as pl
from jax.experimental.pallas import tpu as pltpu


def _prepass_body(x_ref, w_ref, asrc_ref, adst_ref, wh_ref, s_ref,
                  dexp_ref, dexp2_ref, dmax_ref):
    i = pl.program_id(0)
    wh = jnp.dot(x_ref[...], w_ref[...], preferred_element_type=jnp.float32)
    wh_ref[...] = wh
    s_ref[...] = jnp.dot(wh, asrc_ref[...], preferred_element_type=jnp.float32)
    d = jnp.dot(wh, adst_ref[...], preferred_element_type=jnp.float32)
    dexp_ref[...] = jnp.exp(d).astype(jnp.bfloat16)
    dexp2_ref[...] = jnp.exp(0.2 * d).astype(jnp.bfloat16)
    bmax = jnp.max(d, axis=0, keepdims=True)

    @pl.when(i == 0)
    def _():
        dmax_ref[...] = bmax

    @pl.when(i > 0)
    def _():
        dmax_ref[...] = jnp.maximum(dmax_ref[...], bmax)


def _main_body(*refs, n, h_heads, d_dim, bm, bn, k_chunks):
    adj_refs = refs[:k_chunks]
    dt_ref, wh2_ref, s_ref, dmax_ref, out_ref, srow_ref = refs[k_chunks:]

    s = s_ref[...]
    x = s + dmax_ref[...]
    m = jnp.maximum(x, 0.2 * x)
    srow_ref[:, :h_heads] = jnp.exp(s - m).astype(jnp.bfloat16)
    srow_ref[:, h_heads:2 * h_heads] = jnp.exp(0.2 * s - m).astype(jnp.bfloat16)

    col_ids = jax.lax.broadcasted_iota(jnp.int32, (1, bn), 1)
    for k in range(k_chunks):
        rows = slice(k * bm, (k + 1) * bm)
        adjm = jnp.where(col_ids < n, adj_refs[k][...], 0.0).astype(jnp.bfloat16)

        for h in range(h_heads):
            u1 = srow_ref[rows, h:h + 1]
            u2 = srow_ref[rows, h_heads + h:h_heads + h + 1]
            v1 = dt_ref[h:h + 1, :]
            v2 = dt_ref[h_heads + h:h_heads + h + 1, :]
            p = jnp.maximum(u1 * v1, u2 * v2) * adjm
            part = jnp.dot(p, wh2_ref[:, 32 * h:32 * h + 32],
                           preferred_element_type=jnp.float32)
            y = part[:, :d_dim] / part[:, d_dim:d_dim + 1]
            out_ref[rows, h * d_dim:(h + 1) * d_dim] = \
                jnp.where(y > 0, y, jnp.exp(y) - 1.0)


def kernel(features, adj, W, a_src, a_dst):
    n, f = features.shape
    h_heads, _, d_dim = W.shape
    hd = h_heads * d_dim

    bm1 = 1000
    r1 = n // bm1
    w_cat = jnp.transpose(W, (1, 0, 2)).reshape(f, hd)
    asrc = jnp.zeros((hd, h_heads), jnp.float32)
    adst = jnp.zeros((hd, h_heads), jnp.float32)
    for h in range(h_heads):
        asrc = asrc.at[h * d_dim:(h + 1) * d_dim, h].set(a_src[h])
        adst = adst.at[h * d_dim:(h + 1) * d_dim, h].set(a_dst[h])

    wh, s, dexp, dexp2, dmax = pl.pallas_call(
        _prepass_body,
        grid=(r1,),
        in_specs=[
            pl.BlockSpec((bm1, f), lambda i: (i, 0)),
            pl.BlockSpec((f, hd), lambda i: (0, 0)),
            pl.BlockSpec((hd, h_heads), lambda i: (0, 0)),
            pl.BlockSpec((hd, h_heads), lambda i: (0, 0)),
        ],
        out_specs=[
            pl.BlockSpec((bm1, hd), lambda i: (i, 0)),
            pl.BlockSpec((bm1, h_heads), lambda i: (i, 0)),
            pl.BlockSpec((bm1, h_heads), lambda i: (i, 0)),
            pl.BlockSpec((bm1, h_heads), lambda i: (i, 0)),
            pl.BlockSpec((1, h_heads), lambda i: (0, 0)),
        ],
        out_shape=[
            jax.ShapeDtypeStruct((n, hd), jnp.float32),
            jax.ShapeDtypeStruct((n, h_heads), jnp.float32),
            jax.ShapeDtypeStruct((n, h_heads), jnp.bfloat16),
            jax.ShapeDtypeStruct((n, h_heads), jnp.bfloat16),
            jax.ShapeDtypeStruct((1, h_heads), jnp.float32),
        ],
    )(features, w_cat, asrc, adst)

    bm, kc = 200, 2
    bn = 128 * pl.cdiv(n, 128)
    npad = bn
    rr = pl.cdiv(n, bm * kc)

    dt_pad = jnp.zeros((8, npad), jnp.bfloat16)
    dt_pad = dt_pad.at[:h_heads, :n].set(dexp.T.astype(jnp.bfloat16))
    dt_pad = dt_pad.at[h_heads:2 * h_heads, :n].set(
        dexp2.T.astype(jnp.bfloat16))
    wh2_pad = jnp.zeros((npad, 128), jnp.bfloat16)
    for h in range(h_heads):
        wh2_pad = wh2_pad.at[:n, 32 * h:32 * h + d_dim].set(
            wh[:, h * d_dim:(h + 1) * d_dim].astype(jnp.bfloat16))
        wh2_pad = wh2_pad.at[:n, 32 * h + d_dim].set(
            jnp.ones((n,), jnp.bfloat16))

    body = functools.partial(_main_body, n=n, h_heads=h_heads, d_dim=d_dim,
                             bm=bm, bn=bn, k_chunks=kc)
    adj_specs = [pl.BlockSpec((bm, bn), lambda r, k=k: (r * kc + k, 0))
                 for k in range(kc)]
    out = pl.pallas_call(
        body,
        grid=(rr,),
        in_specs=adj_specs + [
            pl.BlockSpec((8, bn), lambda r: (0, 0)),
            pl.BlockSpec((bn, 128), lambda r: (0, 0)),
            pl.BlockSpec((bm * kc, h_heads), lambda r: (r, 0)),
            pl.BlockSpec((1, h_heads), lambda r: (0, 0)),
        ],
        out_specs=pl.BlockSpec((bm * kc, hd), lambda r: (r, 0)),
        out_shape=jax.ShapeDtypeStruct((n, hd), jnp.float32),
        scratch_shapes=[
            pltpu.VMEM((bm * kc, 2 * h_heads), jnp.bfloat16),
        ],
        compiler_params=pltpu.CompilerParams(
            dimension_semantics=("parallel",),
        ),
    )(*([adj] * kc), dt_pad, wh2_pad, s, dmax)
    return out


# speedup vs baseline: 1.4623x; 1.0864x over previous
import functools

import jax
import jax.numpy as jnp
from jax.experimental import pallas as pl
from jax.experimental.pallas import tpu as pltpu


def _prepass_body(x_ref, w_ref, asrc_ref, adst_ref, wh_ref, s_ref,
                  dexp_ref, dexp2_ref, dmax_ref):
    i = pl.program_id(0)
    wh = jnp.dot(x_ref[...], w_ref[...], preferred_element_type=jnp.float32)
    wh_ref[...] = wh
    s_ref[...] = jnp.dot(wh, asrc_ref[...], preferred_element_type=jnp.float32)
    d = jnp.dot(wh, adst_ref[...], preferred_element_type=jnp.float32)
    dexp_ref[...] = jnp.exp(d).astype(jnp.bfloat16)
    dexp2_ref[...] = jnp.exp(0.2 * d).astype(jnp.bfloat16)
    bmax = jnp.max(d, axis=0, keepdims=True)

    @pl.when(i == 0)
    def _():
        dmax_ref[...] = bmax

    @pl.when(i > 0)
    def _():
        dmax_ref[...] = jnp.maximum(dmax_ref[...], bmax)


def _main_body(*refs, n, h_heads, d_dim, bm, bn, k_chunks):
    adj_refs = refs[:k_chunks]
    dt_ref, wh2_ref, s_ref, dmax_ref, out_ref, srow_ref = refs[k_chunks:]

    s = s_ref[...]
    x = s + dmax_ref[...]
    m = jnp.maximum(x, 0.2 * x)
    srow_ref[:, :h_heads] = jnp.exp(s - m).astype(jnp.bfloat16)
    srow_ref[:, h_heads:2 * h_heads] = jnp.exp(0.2 * s - m).astype(jnp.bfloat16)

    col_ids = jax.lax.broadcasted_iota(jnp.int32, (1, bn), 1)
    for k in range(k_chunks):
        rows = slice(k * bm, (k + 1) * bm)
        adjm = jnp.where(col_ids < n, adj_refs[k][...], 0.0).astype(jnp.bfloat16)

        for h in range(h_heads):
            u1 = srow_ref[rows, h:h + 1]
            u2 = srow_ref[rows, h_heads + h:h_heads + h + 1]
            v1 = dt_ref[h:h + 1, :]
            v2 = dt_ref[h_heads + h:h_heads + h + 1, :]
            p = jnp.maximum(u1 * v1, u2 * v2) * adjm
            part = jnp.dot(p, wh2_ref[:, 32 * h:32 * h + 32],
                           preferred_element_type=jnp.float32)
            y = part[:, :d_dim] / part[:, d_dim:d_dim + 1]
            out_ref[rows, h * d_dim:(h + 1) * d_dim] = \
                jnp.where(y > 0, y, jnp.exp(y) - 1.0)


def kernel(features, adj, W, a_src, a_dst):
    n, f = features.shape
    h_heads, _, d_dim = W.shape
    hd = h_heads * d_dim

    bm1 = 1000
    r1 = n // bm1
    w_cat = jnp.transpose(W, (1, 0, 2)).reshape(f, hd)
    asrc = jnp.zeros((hd, h_heads), jnp.float32)
    adst = jnp.zeros((hd, h_heads), jnp.float32)
    for h in range(h_heads):
        asrc = asrc.at[h * d_dim:(h + 1) * d_dim, h].set(a_src[h])
        adst = adst.at[h * d_dim:(h + 1) * d_dim, h].set(a_dst[h])

    wh, s, dexp, dexp2, dmax = pl.pallas_call(
        _prepass_body,
        grid=(r1,),
        in_specs=[
            pl.BlockSpec((bm1, f), lambda i: (i, 0)),
            pl.BlockSpec((f, hd), lambda i: (0, 0)),
            pl.BlockSpec((hd, h_heads), lambda i: (0, 0)),
            pl.BlockSpec((hd, h_heads), lambda i: (0, 0)),
        ],
        out_specs=[
            pl.BlockSpec((bm1, hd), lambda i: (i, 0)),
            pl.BlockSpec((bm1, h_heads), lambda i: (i, 0)),
            pl.BlockSpec((bm1, h_heads), lambda i: (i, 0)),
            pl.BlockSpec((bm1, h_heads), lambda i: (i, 0)),
            pl.BlockSpec((1, h_heads), lambda i: (0, 0)),
        ],
        out_shape=[
            jax.ShapeDtypeStruct((n, hd), jnp.float32),
            jax.ShapeDtypeStruct((n, h_heads), jnp.float32),
            jax.ShapeDtypeStruct((n, h_heads), jnp.bfloat16),
            jax.ShapeDtypeStruct((n, h_heads), jnp.bfloat16),
            jax.ShapeDtypeStruct((1, h_heads), jnp.float32),
        ],
    )(features, w_cat, asrc, adst)

    bm, kc = 200, 2
    bn = 128 * pl.cdiv(n, 128)
    npad = bn
    rr = pl.cdiv(n, bm * kc)

    dt_pad = jnp.zeros((8, npad), jnp.bfloat16)
    dt_pad = dt_pad.at[:h_heads, :n].set(dexp.T)
    dt_pad = dt_pad.at[h_heads:2 * h_heads, :n].set(dexp2.T)
    wh2_pad = jnp.zeros((npad, 128), jnp.bfloat16)
    for h in range(h_heads):
        wh2_pad = wh2_pad.at[:n, 32 * h:32 * h + d_dim].set(
            wh[:, h * d_dim:(h + 1) * d_dim])
        wh2_pad = wh2_pad.at[:n, 32 * h + d_dim].set(1.0)

    body = functools.partial(_main_body, n=n, h_heads=h_heads, d_dim=d_dim,
                             bm=bm, bn=bn, k_chunks=kc)
    adj_specs = [pl.BlockSpec((bm, bn), lambda r, k=k: (r * kc + k, 0))
                 for k in range(kc)]
    out = pl.pallas_call(
        body,
        grid=(rr,),
        in_specs=adj_specs + [
            pl.BlockSpec((8, bn), lambda r: (0, 0)),
            pl.BlockSpec((bn, 128), lambda r: (0, 0)),
            pl.BlockSpec((bm * kc, h_heads), lambda r: (r, 0)),
            pl.BlockSpec((1, h_heads), lambda r: (0, 0)),
        ],
        out_specs=pl.BlockSpec((bm * kc, hd), lambda r: (r, 0)),
        out_shape=jax.ShapeDtypeStruct((n, hd), jnp.float32),
        scratch_shapes=[
            pltpu.VMEM((bm * kc, 2 * h_heads), jnp.bfloat16),
        ],
        compiler_params=pltpu.CompilerParams(
            dimension_semantics=("parallel",),
        ),
    )(*([adj] * kc), dt_pad, wh2_pad, s, dmax)
    return out
